# baseline (device time: 37734 ns/iter reference)
import jax
import jax.numpy as jnp
from jax import lax
from jax.experimental import pallas as pl
from jax.experimental.pallas import tpu as pltpu

N_DEV = 16
M = 768
N = 768
CHUNK = M // N_DEV
NH = 3
HALF = CHUNK // NH


def kernel(A, B):
    def body(a_ref, b_ref, out_ref, pb_ref, rs_buf,
             rs_send_sems, rs_recv_sems, ag_send_sems, ag_recv_sems):
        my = lax.axis_index("i")

        a_bf = a_ref[...].astype(jnp.bfloat16)
        b_bf = b_ref[...].astype(jnp.bfloat16)
        pb_ref[...] = jnp.dot(a_bf, b_bf,
                              preferred_element_type=jnp.float32
                              ).astype(jnp.bfloat16)

        barrier_sem = pltpu.get_barrier_semaphore()
        for j in range(1, N_DEV):
            nbr = lax.rem(my + j, N_DEV)
            pl.semaphore_signal(barrier_sem, inc=1, device_id=(nbr,),
                                device_id_type=pl.DeviceIdType.MESH)
        pl.semaphore_wait(barrier_sem, N_DEV - 1)

        rs_rdmas = {}
        for h in range(NH):
            for j in range(1, N_DEV):
                tgt = lax.rem(my + j, N_DEV)
                rdma = pltpu.make_async_remote_copy(
                    src_ref=pb_ref.at[pl.ds(tgt * CHUNK + h * HALF, HALF), :],
                    dst_ref=rs_buf.at[h, j],
                    send_sem=rs_send_sems.at[h, j],
                    recv_sem=rs_recv_sems.at[h, j],
                    device_id=(tgt,),
                    device_id_type=pl.DeviceIdType.MESH,
                )
                rdma.start()
                rs_rdmas[h, j] = rdma

        ag_rdmas = {}
        for h in range(NH):
            rows = pl.ds(my * CHUNK + h * HALF, HALF)
            acc = pb_ref[rows, :].astype(jnp.float32)
            for j in range(1, N_DEV):
                rs_rdmas[h, j].wait_recv()
                acc = acc + rs_buf[h, j].astype(jnp.float32)
            out_ref[rows, :] = acc.astype(jnp.bfloat16)
            for j in range(1, N_DEV):
                tgt = lax.rem(my + j, N_DEV)
                rdma = pltpu.make_async_remote_copy(
                    src_ref=out_ref.at[rows, :],
                    dst_ref=out_ref.at[rows, :],
                    send_sem=ag_send_sems.at[h, j],
                    recv_sem=ag_recv_sems.at[h, j],
                    device_id=(tgt,),
                    device_id_type=pl.DeviceIdType.MESH,
                )
                rdma.start()
                ag_rdmas[h, j] = rdma

        for r in ag_rdmas.values():
            r.wait_recv()
        for r in rs_rdmas.values():
            r.wait_send()
        for r in ag_rdmas.values():
            r.wait_send()

    return pl.pallas_call(
        body,
        out_shape=jax.ShapeDtypeStruct((M, N), jnp.bfloat16),
        in_specs=[pl.BlockSpec(memory_space=pltpu.VMEM),
                  pl.BlockSpec(memory_space=pltpu.VMEM)],
        out_specs=pl.BlockSpec(memory_space=pltpu.VMEM),
        scratch_shapes=[
            pltpu.VMEM((M, N), jnp.bfloat16),
            pltpu.VMEM((NH, N_DEV, HALF, N), jnp.bfloat16),
            pltpu.SemaphoreType.DMA((NH, N_DEV)),
            pltpu.SemaphoreType.DMA((NH, N_DEV)),
            pltpu.SemaphoreType.DMA((NH, N_DEV)),
            pltpu.SemaphoreType.DMA((NH, N_DEV)),
        ],
        compiler_params=pltpu.CompilerParams(collective_id=0),
    )(A, B)


# device time: 35333 ns/iter; 1.0680x vs baseline; 1.0680x over previous
import jax
import jax.numpy as jnp
from jax import lax
from jax.experimental import pallas as pl
from jax.experimental.pallas import tpu as pltpu

N_DEV = 16
M = 768
N = 768
CHUNK = M // N_DEV
NH = 2
HALF = CHUNK // NH


def kernel(A, B):
    def body(a_ref, b_ref, out_ref, pb_ref, rs_buf,
             rs_send_sems, rs_recv_sems, ag_send_sems, ag_recv_sems):
        my = lax.axis_index("i")

        barrier_sem = pltpu.get_barrier_semaphore()
        for j in range(1, N_DEV):
            nbr = lax.rem(my + j, N_DEV)
            pl.semaphore_signal(barrier_sem, inc=1, device_id=(nbr,),
                                device_id_type=pl.DeviceIdType.MESH)

        a_bf = a_ref[...].astype(jnp.bfloat16)
        b_bf = b_ref[...].astype(jnp.bfloat16)
        pb_ref[...] = jnp.dot(a_bf, b_bf,
                              preferred_element_type=jnp.float32
                              ).astype(jnp.bfloat16)

        pl.semaphore_wait(barrier_sem, N_DEV - 1)

        rs_rdmas = {}
        for h in range(NH):
            for j in range(1, N_DEV):
                tgt = lax.rem(my + j, N_DEV)
                rdma = pltpu.make_async_remote_copy(
                    src_ref=pb_ref.at[pl.ds(tgt * CHUNK + h * HALF, HALF), :],
                    dst_ref=rs_buf.at[h, j],
                    send_sem=rs_send_sems.at[h, j],
                    recv_sem=rs_recv_sems.at[h, j],
                    device_id=(tgt,),
                    device_id_type=pl.DeviceIdType.MESH,
                )
                rdma.start()
                rs_rdmas[h, j] = rdma

        ag_rdmas = {}
        for h in range(NH):
            rows = pl.ds(my * CHUNK + h * HALF, HALF)
            acc = pb_ref[rows, :].astype(jnp.float32)
            for j in range(1, N_DEV):
                rs_rdmas[h, j].wait_recv()
                acc = acc + rs_buf[h, j].astype(jnp.float32)
            out_ref[rows, :] = acc.astype(jnp.bfloat16)
            for j in range(1, N_DEV):
                tgt = lax.rem(my + j, N_DEV)
                rdma = pltpu.make_async_remote_copy(
                    src_ref=out_ref.at[rows, :],
                    dst_ref=out_ref.at[rows, :],
                    send_sem=ag_send_sems.at[h, j],
                    recv_sem=ag_recv_sems.at[h, j],
                    device_id=(tgt,),
                    device_id_type=pl.DeviceIdType.MESH,
                )
                rdma.start()
                ag_rdmas[h, j] = rdma

        for r in ag_rdmas.values():
            r.wait_recv()
        for r in rs_rdmas.values():
            r.wait_send()
        for r in ag_rdmas.values():
            r.wait_send()

    return pl.pallas_call(
        body,
        out_shape=jax.ShapeDtypeStruct((M, N), jnp.bfloat16),
        in_specs=[pl.BlockSpec(memory_space=pltpu.VMEM),
                  pl.BlockSpec(memory_space=pltpu.VMEM)],
        out_specs=pl.BlockSpec(memory_space=pltpu.VMEM),
        scratch_shapes=[
            pltpu.VMEM((M, N), jnp.bfloat16),
            pltpu.VMEM((NH, N_DEV, HALF, N), jnp.bfloat16),
            pltpu.SemaphoreType.DMA((NH, N_DEV)),
            pltpu.SemaphoreType.DMA((NH, N_DEV)),
            pltpu.SemaphoreType.DMA((NH, N_DEV)),
            pltpu.SemaphoreType.DMA((NH, N_DEV)),
        ],
        compiler_params=pltpu.CompilerParams(collective_id=0),
    )(A, B)
